# baseline (speedup 1.0000x reference)
import functools

import jax
import jax.numpy as jnp
from jax.experimental import pallas as pl
from jax.experimental.pallas import tpu as pltpu

_PAD = 3
_KSIZE = 3
_BN_EPS = 1e-5


def _fused_kernel(x_ref, wv_ref, wh_ref, b_ref, g_ref, bt_ref, o_ref,
                  scr_ref, feat_ref, stats_ref, *,
                  offs_v, offs_h, N, H, W, WG):
    M = H * WG
    Cin = x_ref.shape[3]
    j = pl.program_id(0)

    top = _PAD * WG

    @pl.when(j == 0)
    def _init():
        stats_ref[...] = jnp.zeros_like(stats_ref)
        scr_ref[0:top, :] = jnp.zeros((top, Cin), jnp.bfloat16)
        scr_ref[top + M:, :] = jnp.zeros((scr_ref.shape[0] - top - M, Cin),
                                         jnp.bfloat16)

    @pl.when(j < N)
    def _conv_phase():
        xr = x_ref[0].astype(jnp.bfloat16)
        zl = jnp.zeros((H, _PAD, Cin), jnp.bfloat16)
        zr = jnp.zeros((H, WG - W - _PAD, Cin), jnp.bfloat16)
        xrow = jnp.concatenate([zl, xr, zr], axis=1).reshape(M, Cin)
        scr_ref[top:top + M, :] = xrow
        xv = jnp.concatenate([scr_ref[o:o + M, :] for o in offs_v], axis=1)
        xh = jnp.concatenate([scr_ref[o:o + M, :] for o in offs_h], axis=1)
        yv = jnp.dot(xv, wv_ref[...], preferred_element_type=jnp.float32)
        yh = jnp.dot(xh, wh_ref[...], preferred_element_type=jnp.float32)
        ym = jnp.concatenate([yv, yh], axis=1)
        ys = (ym.reshape(H, WG, -1)[:, _PAD:_PAD + W, :]
              + b_ref[...].reshape(1, 1, -1))
        feat_ref[j] = ys.reshape(H * W, -1).astype(feat_ref.dtype)
        ssum = jnp.sum(ys, axis=(0, 1)).reshape(1, -1)
        ssq = jnp.sum(ys * ys, axis=(0, 1)).reshape(1, -1)
        stats_ref[...] += jnp.concatenate([ssum, ssq], axis=0)

    @pl.when(j >= N)
    def _bn_phase():
        B = o_ref.shape[0]
        inv_count = 1.0 / float(N * H * W)
        tot = stats_ref[...]
        mean = tot[0:1] * inv_count
        var = tot[1:2] * inv_count - mean * mean
        scale = g_ref[...] * jax.lax.rsqrt(var + jnp.float32(_BN_EPS))
        shift = bt_ref[...] - mean * scale
        fb = feat_ref[pl.ds(jnp.maximum(j - N, 0) * B, B)].astype(jnp.float32)
        z = jnp.maximum(fb * scale.reshape(1, 1, -1) + shift.reshape(1, 1, -1),
                        0.0)
        o_ref[...] = z.reshape(B, H, W, -1)


def _pack_taps(ws, dils, Cin):
    cols = []
    for w, dil in zip(ws, dils):
        c = w.shape[0]
        w2 = w.reshape(c, Cin, _KSIZE).astype(jnp.float32)
        zero = jnp.zeros((Cin, c), jnp.float32)
        slots = {3 + (t - 1) * dil: w2[:, :, t].T for t in range(_KSIZE)}
        cols.append(jnp.concatenate([slots.get(s, zero) for s in range(7)],
                                    axis=0))
    return jnp.concatenate(cols, axis=1)


def kernel(x, w_first, w_second, w_third, w_first2, w_second2, w_third2,
           b_first, b_second, b_third, b_first2, b_second2, b_third2,
           gamma, beta):
    N, Cin, H, W = x.shape
    Cout = gamma.shape[0]

    WG = W + 2 * _PAD + 2
    HP = H + 2 * _PAD
    M = H * WG
    R = HP * WG

    xt = jnp.transpose(x, (0, 2, 3, 1))

    offs_v = tuple((_PAD + dh) * WG for dh in range(-3, 4))
    offs_h = tuple(_PAD * WG + dw for dw in range(-3, 4))

    wv = _pack_taps([w_first, w_second, w_third], (1, 2, 3), Cin).astype(jnp.bfloat16)
    wh = _pack_taps([w_first2, w_second2, w_third2], (1, 2, 3), Cin).astype(jnp.bfloat16)
    bias = jnp.concatenate([b_first, b_second, b_third,
                            b_first2, b_second2, b_third2])
    bias2 = bias.reshape(1, Cout).astype(jnp.float32)
    gamma2 = gamma.reshape(1, Cout).astype(jnp.float32)
    beta2 = beta.reshape(1, Cout).astype(jnp.float32)

    fused_fn = functools.partial(_fused_kernel, offs_v=offs_v, offs_h=offs_h,
                                 N=N, H=H, W=W, WG=WG)
    B = next(b for b in (8, 4, 2, 1) if N % b == 0)
    out = pl.pallas_call(
        fused_fn,
        out_shape=jax.ShapeDtypeStruct((N, H, W, Cout), jnp.float32),
        grid=(N + N // B,),
        in_specs=[pl.BlockSpec((1, H, W, Cin),
                               lambda j: (jnp.minimum(j, N - 1), 0, 0, 0)),
                  pl.BlockSpec(wv.shape, lambda j: (0, 0)),
                  pl.BlockSpec(wh.shape, lambda j: (0, 0)),
                  pl.BlockSpec((1, Cout), lambda j: (0, 0)),
                  pl.BlockSpec((1, Cout), lambda j: (0, 0)),
                  pl.BlockSpec((1, Cout), lambda j: (0, 0))],
        out_specs=pl.BlockSpec((B, H, W, Cout),
                               lambda j: (jnp.maximum(j - N, 0), 0, 0, 0)),
        scratch_shapes=[pltpu.VMEM((R, Cin), jnp.bfloat16),
                        pltpu.VMEM((N, H * W, Cout), jnp.bfloat16),
                        pltpu.VMEM((2, Cout), jnp.float32)],
        compiler_params=pltpu.CompilerParams(
            dimension_semantics=("arbitrary",)),
    )(xt, wv, wh, bias2, gamma2, beta2)
    return jnp.transpose(out, (0, 3, 1, 2))


# speedup vs baseline: 1.0016x; 1.0016x over previous
import functools

import jax
import jax.numpy as jnp
from jax.experimental import pallas as pl
from jax.experimental.pallas import tpu as pltpu

_PAD = 3
_KSIZE = 3
_BN_EPS = 1e-5


def _fused_kernel(x_ref, wv_ref, wh_ref, b_ref, g_ref, bt_ref, o_ref,
                  scr_ref, feat_ref, stats_ref, *,
                  offs_v, offs_h, N, H, W, WG):
    M = H * WG
    Cin = x_ref.shape[3]
    j = pl.program_id(0)

    top = _PAD * WG

    @pl.when(j == 0)
    def _init():
        stats_ref[...] = jnp.zeros_like(stats_ref)
        scr_ref[0:top, :] = jnp.zeros((top, Cin), jnp.bfloat16)
        scr_ref[top + M:, :] = jnp.zeros((scr_ref.shape[0] - top - M, Cin),
                                         jnp.bfloat16)

    @pl.when(j < N)
    def _conv_phase():
        xr = x_ref[0].astype(jnp.bfloat16)
        zl = jnp.zeros((H, _PAD, Cin), jnp.bfloat16)
        zr = jnp.zeros((H, WG - W - _PAD, Cin), jnp.bfloat16)
        xrow = jnp.concatenate([zl, xr, zr], axis=1).reshape(M, Cin)
        scr_ref[top:top + M, :] = xrow
        xv = jnp.concatenate([scr_ref[o:o + M, :] for o in offs_v], axis=1)
        xh = jnp.concatenate([scr_ref[o:o + M, :] for o in offs_h], axis=1)
        yv = jnp.dot(xv, wv_ref[...], preferred_element_type=jnp.float32)
        yh = jnp.dot(xh, wh_ref[...], preferred_element_type=jnp.float32)
        ym = jnp.concatenate([yv, yh], axis=1)
        ys = (ym.reshape(H, WG, -1)[:, _PAD:_PAD + W, :]
              + b_ref[...].reshape(1, 1, -1))
        feat_ref[j] = ys.reshape(H * W, -1).astype(feat_ref.dtype)
        ssum = jnp.sum(ys, axis=(0, 1)).reshape(1, -1)
        ssq = jnp.sum(ys * ys, axis=(0, 1)).reshape(1, -1)
        stats_ref[...] += jnp.concatenate([ssum, ssq], axis=0)

    @pl.when(j >= N)
    def _bn_phase():
        B = o_ref.shape[0]
        inv_count = 1.0 / float(N * H * W)
        tot = stats_ref[...]
        mean = tot[0:1] * inv_count
        var = tot[1:2] * inv_count - mean * mean
        scale = g_ref[...] * jax.lax.rsqrt(var + jnp.float32(_BN_EPS))
        shift = bt_ref[...] - mean * scale
        fb = feat_ref[pl.ds(jnp.maximum(j - N, 0) * B, B)].astype(jnp.float32)
        z = jnp.maximum(fb * scale.reshape(1, 1, -1) + shift.reshape(1, 1, -1),
                        0.0)
        o_ref[...] = z.reshape(B, H, W, -1)


def _pack_taps(ws, dils, Cin):
    cols = []
    for w, dil in zip(ws, dils):
        c = w.shape[0]
        w2 = w.reshape(c, Cin, _KSIZE).astype(jnp.float32)
        zero = jnp.zeros((Cin, c), jnp.float32)
        slots = {3 + (t - 1) * dil: w2[:, :, t].T for t in range(_KSIZE)}
        cols.append(jnp.concatenate([slots.get(s, zero) for s in range(7)],
                                    axis=0))
    return jnp.concatenate(cols, axis=1)


def kernel(x, w_first, w_second, w_third, w_first2, w_second2, w_third2,
           b_first, b_second, b_third, b_first2, b_second2, b_third2,
           gamma, beta):
    N, Cin, H, W = x.shape
    Cout = gamma.shape[0]

    WG = W + 2 * _PAD + 2
    HP = H + 2 * _PAD
    M = H * WG
    R = HP * WG

    xt = jnp.transpose(x, (0, 2, 3, 1))

    offs_v = tuple((_PAD + dh) * WG for dh in range(-3, 4))
    offs_h = tuple(_PAD * WG + dw for dw in range(-3, 4))

    wv = _pack_taps([w_first, w_second, w_third], (1, 2, 3), Cin).astype(jnp.bfloat16)
    wh = _pack_taps([w_first2, w_second2, w_third2], (1, 2, 3), Cin).astype(jnp.bfloat16)
    bias = jnp.concatenate([b_first, b_second, b_third,
                            b_first2, b_second2, b_third2])
    bias2 = bias.reshape(1, Cout).astype(jnp.float32)
    gamma2 = gamma.reshape(1, Cout).astype(jnp.float32)
    beta2 = beta.reshape(1, Cout).astype(jnp.float32)

    fused_fn = functools.partial(_fused_kernel, offs_v=offs_v, offs_h=offs_h,
                                 N=N, H=H, W=W, WG=WG)
    B = next(b for b in (4, 2, 1) if N % b == 0)
    out = pl.pallas_call(
        fused_fn,
        out_shape=jax.ShapeDtypeStruct((N, H, W, Cout), jnp.float32),
        grid=(N + N // B,),
        in_specs=[pl.BlockSpec((1, H, W, Cin),
                               lambda j: (jnp.minimum(j, N - 1), 0, 0, 0)),
                  pl.BlockSpec(wv.shape, lambda j: (0, 0)),
                  pl.BlockSpec(wh.shape, lambda j: (0, 0)),
                  pl.BlockSpec((1, Cout), lambda j: (0, 0)),
                  pl.BlockSpec((1, Cout), lambda j: (0, 0)),
                  pl.BlockSpec((1, Cout), lambda j: (0, 0))],
        out_specs=pl.BlockSpec((B, H, W, Cout),
                               lambda j: (jnp.maximum(j - N, 0), 0, 0, 0)),
        scratch_shapes=[pltpu.VMEM((R, Cin), jnp.bfloat16),
                        pltpu.VMEM((N, H * W, Cout), jnp.bfloat16),
                        pltpu.VMEM((2, Cout), jnp.float32)],
        compiler_params=pltpu.CompilerParams(
            dimension_semantics=("arbitrary",)),
    )(xt, wv, wh, bias2, gamma2, beta2)
    return jnp.transpose(out, (0, 3, 1, 2))
